# baseline (speedup 1.0000x reference)
import functools

import jax
import jax.numpy as jnp
from jax import lax
from jax.experimental import pallas as pl
from jax.experimental.pallas import tpu as pltpu
from jax.experimental.pallas import tpu_sc as plsc

N_NODES = 10000
N_EDGES = 320000
D = 128
NC = 2
NS = 16
E_TILE = N_EDGES // NS
CH = 80
N_CH = E_TILE // CH
ROWS_A = 624
TAIL_BASE = NS * ROWS_A
TAIL = N_NODES - TAIL_BASE

_mesh = plsc.VectorSubcoreMesh(
    core_axis_name="c", subcore_axis_name="s", num_cores=NC, num_subcores=NS)


@functools.partial(
    pl.kernel,
    out_type=[
        jax.ShapeDtypeStruct((NC, N_NODES, D), jnp.float32),
        jax.ShapeDtypeStruct((NC * N_NODES,), jnp.float32),
    ],
    mesh=_mesh,
    scratch_types=[
        pltpu.VMEM((4, CH, D), jnp.float32),
        pltpu.VMEM((4, CH), jnp.int32),
        pltpu.VMEM((CH,), jnp.float32),
        pltpu.VMEM((ROWS_A,), jnp.float32),
        pltpu.VMEM_SHARED((N_NODES, D), jnp.float32),
        pltpu.VMEM_SHARED((N_NODES,), jnp.float32),
        pltpu.SemaphoreType.DMA,
        pltpu.SemaphoreType.DMA,
        pltpu.SemaphoreType.DMA,
        pltpu.SemaphoreType.DMA,
        pltpu.SemaphoreType.DMA,
        pltpu.SemaphoreType.DMA,
        pltpu.SemaphoreType.DMA,
        pltpu.SemaphoreType.DMA,
    ],
)
def _segment_sums(efeat, eidx, sums_out, cnts_out,
                  feat_buf, idx_buf, ones_buf, zcnt_buf, sum_sh, cnt_sh,
                  lsem0, lsem1, lsem2, lsem3, ssem0, ssem1, ssem2, ssem3):
    cid = lax.axis_index("c")
    sid = lax.axis_index("s")
    lsem = (lsem0, lsem1, lsem2, lsem3)
    ssem = (ssem0, ssem1, ssem2, ssem3)
    z16 = jnp.zeros((16,), jnp.float32)
    o16 = jnp.ones((16,), jnp.float32)

    def _zero_feat_row(i, carry):
        for j in range(D // 16):
            feat_buf[0, i, pl.ds(j * 16, 16)] = z16
        return carry

    lax.fori_loop(0, CH, _zero_feat_row, 0)

    for i in range(ROWS_A // 16):
        zcnt_buf[pl.ds(i * 16, 16)] = z16
    for i in range(CH // 16):
        ones_buf[pl.ds(i * 16, 16)] = o16

    base_r = sid * ROWS_A
    off_z = 0
    while off_z < ROWS_A:
        nz = min(CH, ROWS_A - off_z)
        pltpu.sync_copy(feat_buf.at[0, pl.ds(0, nz)],
                        sum_sh.at[pl.ds(base_r + off_z, nz)])
        off_z += nz
    pltpu.sync_copy(zcnt_buf, cnt_sh.at[pl.ds(base_r, ROWS_A)])

    @pl.when(sid == NS - 1)
    def _():
        pltpu.sync_copy(feat_buf.at[0, pl.ds(0, TAIL)],
                        sum_sh.at[pl.ds(TAIL_BASE, TAIL)])
        pltpu.sync_copy(zcnt_buf.at[pl.ds(0, TAIL)],
                        cnt_sh.at[pl.ds(TAIL_BASE, TAIL)])

    e_base = sid * E_TILE
    N_MAIN = (N_CH // 4) * 4

    def _fire_loads(g, b, idx_row, col0):
        base = e_base + g * CH
        pltpu.async_copy(eidx.at[pl.ds(idx_row * N_EDGES + base, CH)],
                         idx_buf.at[b], lsem[b])
        pltpu.async_copy(efeat.at[pl.ds(base, CH), pl.ds(col0, D)],
                         feat_buf.at[b], lsem[b])

    def _wait_loads(g, b, idx_row, col0):
        base = e_base + g * CH
        pltpu.make_async_copy(eidx.at[pl.ds(idx_row * N_EDGES + base, CH)],
                              idx_buf.at[b], lsem[b]).wait()
        pltpu.make_async_copy(efeat.at[pl.ds(base, CH), pl.ds(col0, D)],
                              feat_buf.at[b], lsem[b]).wait()

    def _fire_scatter(b):
        pltpu.async_copy(feat_buf.at[b], sum_sh.at[idx_buf.at[b]],
                         ssem[b], add=True)
        pltpu.async_copy(ones_buf, cnt_sh.at[idx_buf.at[b]], ssem[b], add=True)

    def _wait_scatter(b):
        pltpu.make_async_copy(feat_buf.at[b], sum_sh.at[idx_buf.at[b]],
                              ssem[b]).wait()
        pltpu.make_async_copy(ones_buf, cnt_sh.at[idx_buf.at[b]],
                              ssem[b]).wait()

    def _core_loop(idx_row, col0):
        _fire_loads(0, 0, idx_row, col0)
        _fire_loads(1, 1, idx_row, col0)
        plsc.subcore_barrier()

        def _body(g2, carry):
            for u in range(4):
                g = g2 * 4 + u
                b = u
                b2 = (u + 2) % 4
                _wait_loads(g, b, idx_row, col0)
                _fire_scatter(b)
                if u < 2:
                    @pl.when(g2 > 0)
                    def _():
                        _wait_scatter(b2)
                else:
                    _wait_scatter(b2)
                _fire_loads(g + 2, b2, idx_row, col0)
            return carry

        lax.fori_loop(0, N_MAIN // 4, _body, 0)

        _wait_loads(N_MAIN, 0, idx_row, col0)
        _fire_scatter(0)
        _wait_scatter(2)
        _wait_loads(N_MAIN + 1, 1, idx_row, col0)
        _fire_scatter(1)
        _wait_scatter(3)
        _wait_scatter(0)
        _wait_scatter(1)

    @pl.when(cid == 0)
    def _():
        _core_loop(1, 0)

    @pl.when(cid == 1)
    def _():
        _core_loop(0, D)

    plsc.subcore_barrier()

    @pl.when(cid == 0)
    def _():
        pltpu.sync_copy(sum_sh.at[pl.ds(base_r, ROWS_A)],
                        sums_out.at[0, pl.ds(base_r, ROWS_A)])
        pltpu.sync_copy(cnt_sh.at[pl.ds(base_r, ROWS_A)], zcnt_buf)
        pltpu.sync_copy(zcnt_buf, cnts_out.at[pl.ds(base_r, ROWS_A)])

        @pl.when(sid == NS - 1)
        def _():
            pltpu.sync_copy(sum_sh.at[pl.ds(TAIL_BASE, TAIL)],
                            sums_out.at[0, pl.ds(TAIL_BASE, TAIL)])
            pltpu.sync_copy(cnt_sh.at[pl.ds(TAIL_BASE, TAIL)],
                            zcnt_buf.at[pl.ds(0, TAIL)])
            pltpu.sync_copy(zcnt_buf.at[pl.ds(0, TAIL)],
                            cnts_out.at[pl.ds(TAIL_BASE, TAIL)])

    @pl.when(cid == 1)
    def _():
        pltpu.sync_copy(sum_sh.at[pl.ds(base_r, ROWS_A)],
                        sums_out.at[1, pl.ds(base_r, ROWS_A)])
        pltpu.sync_copy(cnt_sh.at[pl.ds(base_r, ROWS_A)], zcnt_buf)
        pltpu.sync_copy(zcnt_buf, cnts_out.at[pl.ds(N_NODES + base_r, ROWS_A)])

        @pl.when(sid == NS - 1)
        def _():
            pltpu.sync_copy(sum_sh.at[pl.ds(TAIL_BASE, TAIL)],
                            sums_out.at[1, pl.ds(TAIL_BASE, TAIL)])
            pltpu.sync_copy(cnt_sh.at[pl.ds(TAIL_BASE, TAIL)],
                            zcnt_buf.at[pl.ds(0, TAIL)])
            pltpu.sync_copy(zcnt_buf.at[pl.ds(0, TAIL)],
                            cnts_out.at[pl.ds(N_NODES + TAIL_BASE, TAIL)])


def _tc_heads_body(sums_ref, cnts_ref, wh_ref, bh_ref, wt_ref, bt_ref,
                   nh_ref, nt_ref):
    rec = 1.0 / jnp.maximum(cnts_ref[...], 1.0)
    recb = lax.broadcast_in_dim(rec, (NC, N_NODES, D), (0, 1))
    nf = 0.5 * (sums_ref[0] * recb[0] + sums_ref[1] * recb[1])
    nh_ref[...] = jnp.dot(nf, wh_ref[...],
                          preferred_element_type=jnp.float32) + bh_ref[...]
    nt_ref[...] = jnp.dot(nf, wt_ref[...],
                          preferred_element_type=jnp.float32) + bt_ref[...]


_tc_heads = pl.pallas_call(
    _tc_heads_body,
    out_shape=[jax.ShapeDtypeStruct((N_NODES, D), jnp.float32),
               jax.ShapeDtypeStruct((N_NODES, D), jnp.float32)],
)


@functools.partial(
    pl.kernel,
    out_type=jax.ShapeDtypeStruct((N_EDGES, 2 * D), jnp.float32),
    mesh=_mesh,
    scratch_types=[
        pltpu.VMEM((3, CH), jnp.int32),
        pltpu.VMEM((3, CH, D), jnp.float32),
        pltpu.VMEM_SHARED((N_NODES, D), jnp.float32),
        pltpu.SemaphoreType.DMA,
        pltpu.SemaphoreType.DMA,
        pltpu.SemaphoreType.DMA,
        pltpu.SemaphoreType.DMA,
        pltpu.SemaphoreType.DMA,
        pltpu.SemaphoreType.DMA,
        pltpu.SemaphoreType.DMA,
        pltpu.SemaphoreType.DMA,
        pltpu.SemaphoreType.DMA,
    ],
)
def _gather_heads(nh, nt, eidx, out, idx_buf, rows_buf, tab_sh,
                  lsem0, lsem1, lsem2, wsem0, wsem1, wsem2,
                  gsem0, gsem1, gsem2):
    cid = lax.axis_index("c")
    sid = lax.axis_index("s")
    lsem = (lsem0, lsem1, lsem2)
    wsem = (wsem0, wsem1, wsem2)
    gsem = (gsem0, gsem1, gsem2)
    base_r = sid * ROWS_A
    e_base = sid * E_TILE
    N_MAIN = (N_CH // 3) * 3

    def _fire_idx(g, b, idx_row):
        pltpu.async_copy(eidx.at[pl.ds(idx_row * N_EDGES + e_base + g * CH, CH)],
                         idx_buf.at[b], lsem[b])

    def _wait_idx(g, b, idx_row):
        pltpu.make_async_copy(
            eidx.at[pl.ds(idx_row * N_EDGES + e_base + g * CH, CH)],
            idx_buf.at[b], lsem[b]).wait()

    def _fire_store(g, b, col0):
        pltpu.async_copy(rows_buf.at[b],
                         out.at[pl.ds(e_base + g * CH, CH), pl.ds(col0, D)],
                         wsem[b])

    def _wait_store(g, b, col0):
        pltpu.make_async_copy(rows_buf.at[b],
                              out.at[pl.ds(e_base + g * CH, CH), pl.ds(col0, D)],
                              wsem[b]).wait()

    def _stage_and_loop(tab, idx_row, col0):
        _fire_idx(0, 0, idx_row)
        _fire_idx(1, 1, idx_row)
        pltpu.sync_copy(tab.at[pl.ds(base_r, ROWS_A)],
                        tab_sh.at[pl.ds(base_r, ROWS_A)])

        @pl.when(sid == NS - 1)
        def _():
            pltpu.sync_copy(tab.at[pl.ds(TAIL_BASE, TAIL)],
                            tab_sh.at[pl.ds(TAIL_BASE, TAIL)])

        plsc.subcore_barrier()

        def _body(g2, carry):
            for u in range(3):
                g = g2 * 3 + u
                b = u
                bm1 = (u - 1) % 3
                _wait_idx(g, b, idx_row)
                @pl.when(g2 > 0)
                def _():
                    _wait_store(g - 3, b, col0)

                pltpu.async_copy(tab_sh.at[idx_buf.at[b]], rows_buf.at[b],
                                 gsem[b])
                if u == 0:
                    @pl.when(g2 > 0)
                    def _():
                        pltpu.make_async_copy(tab_sh.at[idx_buf.at[bm1]],
                                              rows_buf.at[bm1],
                                              gsem[bm1]).wait()
                        _fire_store(g - 1, bm1, col0)
                    _fire_idx(g + 2, bm1, idx_row)
                else:
                    pltpu.make_async_copy(tab_sh.at[idx_buf.at[bm1]],
                                          rows_buf.at[bm1], gsem[bm1]).wait()
                    _fire_store(g - 1, bm1, col0)
                    if u == 2:
                        @pl.when(g2 < N_MAIN // 3 - 1)
                        def _():
                            _fire_idx(g + 2, bm1, idx_row)
                    else:
                        _fire_idx(g + 2, bm1, idx_row)
            return carry

        lax.fori_loop(0, N_MAIN // 3, _body, 0)

        _wait_idx(N_MAIN, 0, idx_row)
        _wait_store(N_MAIN - 3, 0, col0)
        pltpu.async_copy(tab_sh.at[idx_buf.at[0]], rows_buf.at[0], gsem[0])
        pltpu.make_async_copy(tab_sh.at[idx_buf.at[2]], rows_buf.at[2],
                              gsem[2]).wait()
        _fire_store(N_MAIN - 1, 2, col0)
        pltpu.make_async_copy(tab_sh.at[idx_buf.at[0]], rows_buf.at[0],
                              gsem[0]).wait()
        _fire_store(N_MAIN, 0, col0)
        _wait_store(N_MAIN - 2, 1, col0)
        _wait_store(N_MAIN - 1, 2, col0)
        _wait_store(N_MAIN, 0, col0)

    @pl.when(cid == 0)
    def _():
        _stage_and_loop(nh, 0, 0)

    @pl.when(cid == 1)
    def _():
        _stage_and_loop(nt, 1, D)


def kernel(efeat, edge_index, W_head, b_head, W_tail, b_tail):
    eidx_flat = edge_index.reshape(-1)
    sums, cnts = _segment_sums(efeat, eidx_flat)
    nh, nt = _tc_heads(sums, cnts.reshape(NC, N_NODES),
                       W_head, b_head.reshape(1, D),
                       W_tail, b_tail.reshape(1, D))
    return _gather_heads(nh, nt, eidx_flat)


# speedup vs baseline: 1.0831x; 1.0831x over previous
import functools

import jax
import jax.numpy as jnp
from jax import lax
from jax.experimental import pallas as pl
from jax.experimental.pallas import tpu as pltpu
from jax.experimental.pallas import tpu_sc as plsc

N_NODES = 10000
N_EDGES = 320000
D = 128
NC = 2
NS = 16
E_TILE = N_EDGES // NS
CH = 80
N_CH = E_TILE // CH
ROWS_A = 624
TAIL_BASE = NS * ROWS_A
TAIL = N_NODES - TAIL_BASE

_mesh = plsc.VectorSubcoreMesh(
    core_axis_name="c", subcore_axis_name="s", num_cores=NC, num_subcores=NS)


@functools.partial(
    pl.kernel,
    out_type=[
        jax.ShapeDtypeStruct((NC, N_NODES, D), jnp.float32),
        jax.ShapeDtypeStruct((NC * N_NODES,), jnp.float32),
    ],
    mesh=_mesh,
    scratch_types=[
        pltpu.VMEM((3, CH, D), jnp.float32),
        pltpu.VMEM((3, CH), jnp.int32),
        pltpu.VMEM((CH,), jnp.float32),
        pltpu.VMEM((ROWS_A,), jnp.float32),
        pltpu.VMEM_SHARED((N_NODES, D), jnp.float32),
        pltpu.VMEM_SHARED((N_NODES,), jnp.float32),
        pltpu.SemaphoreType.DMA,
        pltpu.SemaphoreType.DMA,
        pltpu.SemaphoreType.DMA,
        pltpu.SemaphoreType.DMA,
        pltpu.SemaphoreType.DMA,
        pltpu.SemaphoreType.DMA,
    ],
)
def _segment_sums(efeat, eidx, sums_out, cnts_out,
                  feat_buf, idx_buf, ones_buf, zcnt_buf, sum_sh, cnt_sh,
                  lsem0, lsem1, lsem2, ssem0, ssem1, ssem2):
    cid = lax.axis_index("c")
    sid = lax.axis_index("s")
    lsem = (lsem0, lsem1, lsem2)
    ssem = (ssem0, ssem1, ssem2)
    z16 = jnp.zeros((16,), jnp.float32)
    o16 = jnp.ones((16,), jnp.float32)

    def _zero_feat_row(i, carry):
        for j in range(D // 16):
            feat_buf[0, i, pl.ds(j * 16, 16)] = z16
        return carry

    lax.fori_loop(0, CH, _zero_feat_row, 0)

    for i in range(ROWS_A // 16):
        zcnt_buf[pl.ds(i * 16, 16)] = z16
    for i in range(CH // 16):
        ones_buf[pl.ds(i * 16, 16)] = o16

    base_r = sid * ROWS_A
    off_z = 0
    while off_z < ROWS_A:
        nz = min(CH, ROWS_A - off_z)
        pltpu.sync_copy(feat_buf.at[0, pl.ds(0, nz)],
                        sum_sh.at[pl.ds(base_r + off_z, nz)])
        off_z += nz
    pltpu.sync_copy(zcnt_buf, cnt_sh.at[pl.ds(base_r, ROWS_A)])

    @pl.when(sid == NS - 1)
    def _():
        pltpu.sync_copy(feat_buf.at[0, pl.ds(0, TAIL)],
                        sum_sh.at[pl.ds(TAIL_BASE, TAIL)])
        pltpu.sync_copy(zcnt_buf.at[pl.ds(0, TAIL)],
                        cnt_sh.at[pl.ds(TAIL_BASE, TAIL)])

    e_base = sid * E_TILE
    N_MAIN = (N_CH // 3) * 3

    def _fire_loads(g, b, idx_row, col0):
        base = e_base + g * CH
        pltpu.async_copy(eidx.at[pl.ds(idx_row * N_EDGES + base, CH)],
                         idx_buf.at[b], lsem[b])
        pltpu.async_copy(efeat.at[pl.ds(base, CH), pl.ds(col0, D)],
                         feat_buf.at[b], lsem[b])

    def _wait_loads(g, b, idx_row, col0):
        base = e_base + g * CH
        pltpu.make_async_copy(eidx.at[pl.ds(idx_row * N_EDGES + base, CH)],
                              idx_buf.at[b], lsem[b]).wait()
        pltpu.make_async_copy(efeat.at[pl.ds(base, CH), pl.ds(col0, D)],
                              feat_buf.at[b], lsem[b]).wait()

    def _fire_scatter(b):
        pltpu.async_copy(feat_buf.at[b], sum_sh.at[idx_buf.at[b]],
                         ssem[b], add=True)
        pltpu.async_copy(ones_buf, cnt_sh.at[idx_buf.at[b]], ssem[b], add=True)

    def _wait_scatter(b):
        pltpu.make_async_copy(feat_buf.at[b], sum_sh.at[idx_buf.at[b]],
                              ssem[b]).wait()
        pltpu.make_async_copy(ones_buf, cnt_sh.at[idx_buf.at[b]],
                              ssem[b]).wait()

    def _core_loop(idx_row, col0):
        _fire_loads(0, 0, idx_row, col0)
        _fire_loads(1, 1, idx_row, col0)
        plsc.subcore_barrier()

        def _body(g2, carry):
            for u in range(3):
                g = g2 * 3 + u
                b = u
                bm1 = (u - 1) % 3
                _wait_loads(g, b, idx_row, col0)
                _fire_scatter(b)
                if u == 0:
                    @pl.when(g2 > 0)
                    def _():
                        _wait_scatter(bm1)
                    _fire_loads(g + 2, bm1, idx_row, col0)
                else:
                    _wait_scatter(bm1)
                    if u == 2:
                        @pl.when(g2 < N_MAIN // 3 - 1)
                        def _():
                            _fire_loads(g + 2, bm1, idx_row, col0)
                    else:
                        _fire_loads(g + 2, bm1, idx_row, col0)
            return carry

        lax.fori_loop(0, N_MAIN // 3, _body, 0)

        _wait_loads(N_MAIN, 0, idx_row, col0)
        _fire_scatter(0)
        _wait_scatter(2)
        _wait_scatter(0)

    @pl.when(cid == 0)
    def _():
        _core_loop(1, 0)

    @pl.when(cid == 1)
    def _():
        _core_loop(0, D)

    plsc.subcore_barrier()

    @pl.when(cid == 0)
    def _():
        pltpu.sync_copy(sum_sh.at[pl.ds(base_r, ROWS_A)],
                        sums_out.at[0, pl.ds(base_r, ROWS_A)])
        pltpu.sync_copy(cnt_sh.at[pl.ds(base_r, ROWS_A)], zcnt_buf)
        pltpu.sync_copy(zcnt_buf, cnts_out.at[pl.ds(base_r, ROWS_A)])

        @pl.when(sid == NS - 1)
        def _():
            pltpu.sync_copy(sum_sh.at[pl.ds(TAIL_BASE, TAIL)],
                            sums_out.at[0, pl.ds(TAIL_BASE, TAIL)])
            pltpu.sync_copy(cnt_sh.at[pl.ds(TAIL_BASE, TAIL)],
                            zcnt_buf.at[pl.ds(0, TAIL)])
            pltpu.sync_copy(zcnt_buf.at[pl.ds(0, TAIL)],
                            cnts_out.at[pl.ds(TAIL_BASE, TAIL)])

    @pl.when(cid == 1)
    def _():
        pltpu.sync_copy(sum_sh.at[pl.ds(base_r, ROWS_A)],
                        sums_out.at[1, pl.ds(base_r, ROWS_A)])
        pltpu.sync_copy(cnt_sh.at[pl.ds(base_r, ROWS_A)], zcnt_buf)
        pltpu.sync_copy(zcnt_buf, cnts_out.at[pl.ds(N_NODES + base_r, ROWS_A)])

        @pl.when(sid == NS - 1)
        def _():
            pltpu.sync_copy(sum_sh.at[pl.ds(TAIL_BASE, TAIL)],
                            sums_out.at[1, pl.ds(TAIL_BASE, TAIL)])
            pltpu.sync_copy(cnt_sh.at[pl.ds(TAIL_BASE, TAIL)],
                            zcnt_buf.at[pl.ds(0, TAIL)])
            pltpu.sync_copy(zcnt_buf.at[pl.ds(0, TAIL)],
                            cnts_out.at[pl.ds(N_NODES + TAIL_BASE, TAIL)])


def _tc_heads_body(sums_ref, cnts_ref, wh_ref, bh_ref, wt_ref, bt_ref,
                   nh_ref, nt_ref):
    rec = 1.0 / jnp.maximum(cnts_ref[...], 1.0)
    recb = lax.broadcast_in_dim(rec, (NC, N_NODES, D), (0, 1))
    nf = 0.5 * (sums_ref[0] * recb[0] + sums_ref[1] * recb[1])
    nh_ref[...] = jnp.dot(nf, wh_ref[...],
                          preferred_element_type=jnp.float32) + bh_ref[...]
    nt_ref[...] = jnp.dot(nf, wt_ref[...],
                          preferred_element_type=jnp.float32) + bt_ref[...]


_tc_heads = pl.pallas_call(
    _tc_heads_body,
    out_shape=[jax.ShapeDtypeStruct((N_NODES, D), jnp.float32),
               jax.ShapeDtypeStruct((N_NODES, D), jnp.float32)],
)


@functools.partial(
    pl.kernel,
    out_type=jax.ShapeDtypeStruct((N_EDGES, 2 * D), jnp.float32),
    mesh=_mesh,
    scratch_types=[
        pltpu.VMEM((3, CH), jnp.int32),
        pltpu.VMEM((3, CH, D), jnp.float32),
        pltpu.VMEM_SHARED((N_NODES, D), jnp.float32),
        pltpu.SemaphoreType.DMA,
        pltpu.SemaphoreType.DMA,
        pltpu.SemaphoreType.DMA,
        pltpu.SemaphoreType.DMA,
        pltpu.SemaphoreType.DMA,
        pltpu.SemaphoreType.DMA,
        pltpu.SemaphoreType.DMA,
        pltpu.SemaphoreType.DMA,
        pltpu.SemaphoreType.DMA,
    ],
)
def _gather_heads(nh, nt, eidx, out, idx_buf, rows_buf, tab_sh,
                  lsem0, lsem1, lsem2, wsem0, wsem1, wsem2,
                  gsem0, gsem1, gsem2):
    cid = lax.axis_index("c")
    sid = lax.axis_index("s")
    lsem = (lsem0, lsem1, lsem2)
    wsem = (wsem0, wsem1, wsem2)
    gsem = (gsem0, gsem1, gsem2)
    base_r = sid * ROWS_A
    e_base = sid * E_TILE
    N_MAIN = (N_CH // 3) * 3

    def _fire_idx(g, b, idx_row):
        pltpu.async_copy(eidx.at[pl.ds(idx_row * N_EDGES + e_base + g * CH, CH)],
                         idx_buf.at[b], lsem[b])

    def _wait_idx(g, b, idx_row):
        pltpu.make_async_copy(
            eidx.at[pl.ds(idx_row * N_EDGES + e_base + g * CH, CH)],
            idx_buf.at[b], lsem[b]).wait()

    def _fire_store(g, b, col0):
        pltpu.async_copy(rows_buf.at[b],
                         out.at[pl.ds(e_base + g * CH, CH), pl.ds(col0, D)],
                         wsem[b])

    def _wait_store(g, b, col0):
        pltpu.make_async_copy(rows_buf.at[b],
                              out.at[pl.ds(e_base + g * CH, CH), pl.ds(col0, D)],
                              wsem[b]).wait()

    def _stage_and_loop(tab, idx_row, col0):
        _fire_idx(0, 0, idx_row)
        _fire_idx(1, 1, idx_row)
        pltpu.sync_copy(tab.at[pl.ds(base_r, ROWS_A)],
                        tab_sh.at[pl.ds(base_r, ROWS_A)])

        @pl.when(sid == NS - 1)
        def _():
            pltpu.sync_copy(tab.at[pl.ds(TAIL_BASE, TAIL)],
                            tab_sh.at[pl.ds(TAIL_BASE, TAIL)])

        plsc.subcore_barrier()

        def _body(g2, carry):
            for u in range(3):
                g = g2 * 3 + u
                b = u
                bm1 = (u - 1) % 3
                _wait_idx(g, b, idx_row)
                @pl.when(g2 > 0)
                def _():
                    _wait_store(g - 3, b, col0)

                pltpu.async_copy(tab_sh.at[idx_buf.at[b]], rows_buf.at[b],
                                 gsem[b])
                if u == 0:
                    @pl.when(g2 > 0)
                    def _():
                        pltpu.make_async_copy(tab_sh.at[idx_buf.at[bm1]],
                                              rows_buf.at[bm1],
                                              gsem[bm1]).wait()
                        _fire_store(g - 1, bm1, col0)
                    _fire_idx(g + 2, bm1, idx_row)
                else:
                    pltpu.make_async_copy(tab_sh.at[idx_buf.at[bm1]],
                                          rows_buf.at[bm1], gsem[bm1]).wait()
                    _fire_store(g - 1, bm1, col0)
                    if u == 2:
                        @pl.when(g2 < N_MAIN // 3 - 1)
                        def _():
                            _fire_idx(g + 2, bm1, idx_row)
                    else:
                        _fire_idx(g + 2, bm1, idx_row)
            return carry

        lax.fori_loop(0, N_MAIN // 3, _body, 0)

        _wait_idx(N_MAIN, 0, idx_row)
        _wait_store(N_MAIN - 3, 0, col0)
        pltpu.async_copy(tab_sh.at[idx_buf.at[0]], rows_buf.at[0], gsem[0])
        pltpu.make_async_copy(tab_sh.at[idx_buf.at[2]], rows_buf.at[2],
                              gsem[2]).wait()
        _fire_store(N_MAIN - 1, 2, col0)
        pltpu.make_async_copy(tab_sh.at[idx_buf.at[0]], rows_buf.at[0],
                              gsem[0]).wait()
        _fire_store(N_MAIN, 0, col0)
        _wait_store(N_MAIN - 2, 1, col0)
        _wait_store(N_MAIN - 1, 2, col0)
        _wait_store(N_MAIN, 0, col0)

    @pl.when(cid == 0)
    def _():
        _stage_and_loop(nh, 0, 0)

    @pl.when(cid == 1)
    def _():
        _stage_and_loop(nt, 1, D)


def kernel(efeat, edge_index, W_head, b_head, W_tail, b_tail):
    eidx_flat = edge_index.reshape(-1)
    sums, cnts = _segment_sums(efeat, eidx_flat)
    nh, nt = _tc_heads(sums, cnts.reshape(NC, N_NODES),
                       W_head, b_head.reshape(1, D),
                       W_tail, b_tail.reshape(1, D))
    return _gather_heads(nh, nt, eidx_flat)
